# baseline (speedup 1.0000x reference)
import functools

import jax
import jax.numpy as jnp
import numpy as np
from jax.experimental import pallas as pl


def _mm_body(m_ref, x_ref, o_ref, *, take_abs):
    acc = jnp.dot(m_ref[...], x_ref[...], preferred_element_type=jnp.float32)
    if take_abs:
        acc = jnp.abs(acc)
    o_ref[...] = acc


def _rowblock_matmul(m, x, *, take_abs=False, block_rows=400):
    n, k = m.shape
    d = x.shape[1]
    assert n % block_rows == 0
    return pl.pallas_call(
        functools.partial(_mm_body, take_abs=take_abs),
        grid=(n // block_rows,),
        in_specs=[
            pl.BlockSpec((block_rows, k), lambda i: (i, 0)),
            pl.BlockSpec((k, d), lambda i: (0, 0)),
        ],
        out_specs=pl.BlockSpec((block_rows, d), lambda i: (i, 0)),
        out_shape=jax.ShapeDtypeStruct((n, d), jnp.float32),
    )(m, x)


def _level1_body(a_ref, p1_ref, p2_ref, p3_ref, x_ref,
                 ha_ref, s1_ref, s2_ref, s3_ref, ab_ref):
    m = pl.program_id(0)
    x = x_ref[...]

    @pl.when(m == 0)
    def _():
        a = a_ref[...]
        ha_ref[...] = jnp.dot(a, x, preferred_element_type=jnp.float32)
        ab_ref[...] = a.astype(jnp.bfloat16)

    @pl.when(m == 1)
    def _():
        s1_ref[...] = jnp.abs(jnp.dot(p1_ref[...], x,
                                      preferred_element_type=jnp.float32))

    @pl.when(m == 2)
    def _():
        s2_ref[...] = jnp.abs(jnp.dot(p2_ref[...], x,
                                      preferred_element_type=jnp.float32))

    @pl.when(m == 3)
    def _():
        s3_ref[...] = jnp.abs(jnp.dot(p3_ref[...], x,
                                      preferred_element_type=jnp.float32))


def _level1(A, P1, P2, P3, x, *, block_rows=200):
    n, k = A.shape
    d = x.shape[1]
    assert n % block_rows == 0
    nb = n // block_rows

    def stream_idx(p):
        def idx(m, i):
            return (jnp.where(m < p, 0, jnp.where(m == p, i, nb - 1)), 0)
        return idx

    big_spec = lambda p: pl.BlockSpec((block_rows, k), stream_idx(p))
    out_spec = lambda p: pl.BlockSpec((block_rows, d), stream_idx(p))

    return pl.pallas_call(
        _level1_body,
        grid=(4, nb),
        in_specs=[big_spec(0), big_spec(1), big_spec(2), big_spec(3),
                  pl.BlockSpec((k, d), lambda m, i: (0, 0))],
        out_specs=[out_spec(0), out_spec(1), out_spec(2), out_spec(3),
                   pl.BlockSpec((block_rows, k), stream_idx(0))],
        out_shape=[jax.ShapeDtypeStruct((n, d), jnp.float32)] * 4
                  + [jax.ShapeDtypeStruct((n, k), jnp.bfloat16)],
    )(A, P1, P2, P3, x)


def _epilogue_body(s_pair_ref,
                   h1_ref, h2_ref, h3_ref, h4_ref, h5_ref, h6_ref,
                   p1_ref, p2_ref, p3_ref, p4_ref, p5_ref, p6_ref,
                   alo_ref, ahi_ref, abiglo_ref, abighi_ref,
                   dev_ref, dod_ref, qcat_ref, rstack_ref,
                   hp_ref, att_ref, *, first_half_blocks):
    i = pl.program_id(0)

    dev = dev_ref[...]
    dod = dod_ref[...]

    ts = s_pair_ref[...]
    e1 = jnp.dot(dev, jnp.dot(ts, alo_ref[...], preferred_element_type=jnp.float32),
                 preferred_element_type=jnp.float32)
    e1 = e1 + jnp.dot(dod, jnp.dot(ts, ahi_ref[...], preferred_element_type=jnp.float32),
                      preferred_element_type=jnp.float32)

    tcat = jnp.concatenate([p1_ref[...], p2_ref[...], p3_ref[...],
                            p4_ref[...], p5_ref[...], p6_ref[...]], axis=1)
    e2 = jnp.dot(dev, jnp.dot(tcat, abiglo_ref[...], preferred_element_type=jnp.float32),
                 preferred_element_type=jnp.float32)
    e2 = e2 + jnp.dot(dod, jnp.dot(tcat, abighi_ref[...], preferred_element_type=jnp.float32),
                      preferred_element_type=jnp.float32)

    e = jnp.where(i < first_half_blocks, e1, e2)

    m = jnp.max(e, axis=1, keepdims=True)
    ex = jnp.exp(e - m)
    att = ex / jnp.sum(ex, axis=1, keepdims=True)
    att_ref[...] = att

    hcat = jnp.concatenate([h1_ref[...], h2_ref[...], h3_ref[...],
                            h4_ref[...], h5_ref[...], h6_ref[...]], axis=1)
    attq = jnp.dot(att, qcat_ref[...], preferred_element_type=jnp.float32)
    hp = jnp.dot(hcat * attq, rstack_ref[...], preferred_element_type=jnp.float32)
    hp_ref[...] = hp * jnp.float32(1.0 / 6.0)


def _epilogue(s, hs, avecs, *, block_rows=200):
    n, d = s.shape
    nh = n // 2
    assert nh % block_rows == 0
    nblocks = n // block_rows
    first_half_blocks = nh // block_rows

    a_list = [a.reshape(2 * d) for a in avecs]
    alo = jnp.stack([a[:d] for a in a_list], axis=1)
    ahi = jnp.stack([a[d:] for a in a_list], axis=1)
    abiglo = jnp.zeros((6 * d, 6), jnp.float32)
    abighi = jnp.zeros((6 * d, 6), jnp.float32)
    for c in range(6):
        abiglo = abiglo.at[c * d:(c + 1) * d, c].set(a_list[c][:d])
        abighi = abighi.at[c * d:(c + 1) * d, c].set(a_list[c][d:])
    dev = np.zeros((block_rows, 2 * block_rows), np.float32)
    dod = np.zeros((block_rows, 2 * block_rows), np.float32)
    dev[np.arange(block_rows), 2 * np.arange(block_rows)] = 1.0
    dod[np.arange(block_rows), 2 * np.arange(block_rows) + 1] = 1.0
    qcat = np.zeros((6, 6 * d), np.float32)
    rstack = np.zeros((6 * d, d), np.float32)
    for c in range(6):
        dd = np.arange(d)
        f = 6 * dd + c
        qcat[f // d, c * d + dd] = 1.0
        rstack[c * d + dd, f % d] = 1.0

    fh = first_half_blocks

    def s_pair_idx(i):
        return (jnp.where(i < fh, i, 0), 0)

    def h_pair_idx(i):
        return (jnp.where(i < fh, 0, i - fh), 0)

    def row_idx(i):
        return (i, 0)

    const = lambda i: (0, 0)

    in_specs = (
        [pl.BlockSpec((2 * block_rows, d), s_pair_idx)]
        + [pl.BlockSpec((block_rows, d), row_idx) for _ in range(6)]
        + [pl.BlockSpec((2 * block_rows, d), h_pair_idx) for _ in range(6)]
        + [pl.BlockSpec((d, 6), const), pl.BlockSpec((d, 6), const),
           pl.BlockSpec((6 * d, 6), const), pl.BlockSpec((6 * d, 6), const),
           pl.BlockSpec((block_rows, 2 * block_rows), const),
           pl.BlockSpec((block_rows, 2 * block_rows), const),
           pl.BlockSpec((6, 6 * d), const), pl.BlockSpec((6 * d, d), const)]
    )

    hp, att = pl.pallas_call(
        functools.partial(_epilogue_body, first_half_blocks=fh),
        grid=(nblocks,),
        in_specs=in_specs,
        out_specs=[pl.BlockSpec((block_rows, d), row_idx),
                   pl.BlockSpec((block_rows, 6), row_idx)],
        out_shape=[jax.ShapeDtypeStruct((n, d), jnp.float32),
                   jax.ShapeDtypeStruct((n, 6), jnp.float32)],
    )(s, *hs, *hs,
      alo, ahi, abiglo, abighi, dev, dod, qcat, rstack)
    return hp, att


def kernel(input, A_nor, P_sct1, P_sct2, P_sct3, W, a1, a2, a3, a4, a5, a6):
    n, din = input.shape
    dout = W.shape[1]

    support0 = _rowblock_matmul(input, W, block_rows=1000)

    h_A, h_s1, h_s2, h_s3, A_bf16 = _level1(A_nor, P_sct1, P_sct2, P_sct3,
                                            support0, block_rows=80)
    h_A2 = _rowblock_matmul(A_bf16, h_A.astype(jnp.bfloat16), block_rows=400)
    h_A3 = _rowblock_matmul(A_bf16, h_A2.astype(jnp.bfloat16), block_rows=400)

    hs = (h_A, h_A2, h_A3, h_s1, h_s2, h_s3)
    hp, att = _epilogue(support0, hs, (a1, a2, a3, a4, a5, a6))
    return hp, att.reshape(n, 6, 1)


# speedup vs baseline: 1.3148x; 1.3148x over previous
import functools

import jax
import jax.numpy as jnp
import numpy as np
from jax.experimental import pallas as pl


def _mm_body(m_ref, x_ref, o_ref, *, take_abs):
    acc = jnp.dot(m_ref[...], x_ref[...], preferred_element_type=jnp.float32)
    if take_abs:
        acc = jnp.abs(acc)
    o_ref[...] = acc


def _rowblock_matmul(m, x, *, take_abs=False, block_rows=400):
    n, k = m.shape
    d = x.shape[1]
    assert n % block_rows == 0
    return pl.pallas_call(
        functools.partial(_mm_body, take_abs=take_abs),
        grid=(n // block_rows,),
        in_specs=[
            pl.BlockSpec((block_rows, k), lambda i: (i, 0)),
            pl.BlockSpec((k, d), lambda i: (0, 0)),
        ],
        out_specs=pl.BlockSpec((block_rows, d), lambda i: (i, 0)),
        out_shape=jax.ShapeDtypeStruct((n, d), jnp.float32),
    )(m, x)


def _mm_cast_body(m_ref, x_ref, o_ref, ob_ref):
    m = m_ref[...]
    o_ref[...] = jnp.dot(m, x_ref[...], preferred_element_type=jnp.float32)
    ob_ref[...] = m.astype(jnp.bfloat16)


def _rowblock_matmul_emit_bf16(m, x, *, block_rows=400):
    n, k = m.shape
    d = x.shape[1]
    assert n % block_rows == 0
    return pl.pallas_call(
        _mm_cast_body,
        grid=(n // block_rows,),
        in_specs=[
            pl.BlockSpec((block_rows, k), lambda i: (i, 0)),
            pl.BlockSpec((k, d), lambda i: (0, 0)),
        ],
        out_specs=[pl.BlockSpec((block_rows, d), lambda i: (i, 0)),
                   pl.BlockSpec((block_rows, k), lambda i: (i, 0))],
        out_shape=[jax.ShapeDtypeStruct((n, d), jnp.float32),
                   jax.ShapeDtypeStruct((n, k), jnp.bfloat16)],
    )(m, x)


def _epilogue_body(s_pair_ref,
                   h1_ref, h2_ref, h3_ref, h4_ref, h5_ref, h6_ref,
                   p1_ref, p2_ref, p3_ref, p4_ref, p5_ref, p6_ref,
                   alo_ref, ahi_ref, abiglo_ref, abighi_ref,
                   dev_ref, dod_ref, qcat_ref, rstack_ref,
                   hp_ref, att_ref, *, first_half_blocks):
    i = pl.program_id(0)

    dev = dev_ref[...]
    dod = dod_ref[...]

    ts = s_pair_ref[...]
    e1 = jnp.dot(dev, jnp.dot(ts, alo_ref[...], preferred_element_type=jnp.float32),
                 preferred_element_type=jnp.float32)
    e1 = e1 + jnp.dot(dod, jnp.dot(ts, ahi_ref[...], preferred_element_type=jnp.float32),
                      preferred_element_type=jnp.float32)

    tcat = jnp.concatenate([p1_ref[...], p2_ref[...], p3_ref[...],
                            p4_ref[...], p5_ref[...], p6_ref[...]], axis=1)
    e2 = jnp.dot(dev, jnp.dot(tcat, abiglo_ref[...], preferred_element_type=jnp.float32),
                 preferred_element_type=jnp.float32)
    e2 = e2 + jnp.dot(dod, jnp.dot(tcat, abighi_ref[...], preferred_element_type=jnp.float32),
                      preferred_element_type=jnp.float32)

    e = jnp.where(i < first_half_blocks, e1, e2)

    m = jnp.max(e, axis=1, keepdims=True)
    ex = jnp.exp(e - m)
    att = ex / jnp.sum(ex, axis=1, keepdims=True)
    att_ref[...] = att

    hcat = jnp.concatenate([h1_ref[...], h2_ref[...], h3_ref[...],
                            h4_ref[...], h5_ref[...], h6_ref[...]], axis=1)
    attq = jnp.dot(att, qcat_ref[...], preferred_element_type=jnp.float32)
    hp = jnp.dot(hcat * attq, rstack_ref[...], preferred_element_type=jnp.float32)
    hp_ref[...] = hp * jnp.float32(1.0 / 6.0)


def _epilogue(s, hs, avecs, *, block_rows=200):
    n, d = s.shape
    nh = n // 2
    assert nh % block_rows == 0
    nblocks = n // block_rows
    first_half_blocks = nh // block_rows

    a_list = [a.reshape(2 * d) for a in avecs]
    alo = jnp.stack([a[:d] for a in a_list], axis=1)
    ahi = jnp.stack([a[d:] for a in a_list], axis=1)
    abiglo = jnp.zeros((6 * d, 6), jnp.float32)
    abighi = jnp.zeros((6 * d, 6), jnp.float32)
    for c in range(6):
        abiglo = abiglo.at[c * d:(c + 1) * d, c].set(a_list[c][:d])
        abighi = abighi.at[c * d:(c + 1) * d, c].set(a_list[c][d:])
    dev = np.zeros((block_rows, 2 * block_rows), np.float32)
    dod = np.zeros((block_rows, 2 * block_rows), np.float32)
    dev[np.arange(block_rows), 2 * np.arange(block_rows)] = 1.0
    dod[np.arange(block_rows), 2 * np.arange(block_rows) + 1] = 1.0
    qcat = np.zeros((6, 6 * d), np.float32)
    rstack = np.zeros((6 * d, d), np.float32)
    for c in range(6):
        dd = np.arange(d)
        f = 6 * dd + c
        qcat[f // d, c * d + dd] = 1.0
        rstack[c * d + dd, f % d] = 1.0

    fh = first_half_blocks

    def s_pair_idx(i):
        return (jnp.where(i < fh, i, 0), 0)

    def h_pair_idx(i):
        return (jnp.where(i < fh, 0, i - fh), 0)

    def row_idx(i):
        return (i, 0)

    const = lambda i: (0, 0)

    in_specs = (
        [pl.BlockSpec((2 * block_rows, d), s_pair_idx)]
        + [pl.BlockSpec((block_rows, d), row_idx) for _ in range(6)]
        + [pl.BlockSpec((2 * block_rows, d), h_pair_idx) for _ in range(6)]
        + [pl.BlockSpec((d, 6), const), pl.BlockSpec((d, 6), const),
           pl.BlockSpec((6 * d, 6), const), pl.BlockSpec((6 * d, 6), const),
           pl.BlockSpec((block_rows, 2 * block_rows), const),
           pl.BlockSpec((block_rows, 2 * block_rows), const),
           pl.BlockSpec((6, 6 * d), const), pl.BlockSpec((6 * d, d), const)]
    )

    hp, att = pl.pallas_call(
        functools.partial(_epilogue_body, first_half_blocks=fh),
        grid=(nblocks,),
        in_specs=in_specs,
        out_specs=[pl.BlockSpec((block_rows, d), row_idx),
                   pl.BlockSpec((block_rows, 6), row_idx)],
        out_shape=[jax.ShapeDtypeStruct((n, d), jnp.float32),
                   jax.ShapeDtypeStruct((n, 6), jnp.float32)],
    )(s, *hs, *hs,
      alo, ahi, abiglo, abighi, dev, dod, qcat, rstack)
    return hp, att


def kernel(input, A_nor, P_sct1, P_sct2, P_sct3, W, a1, a2, a3, a4, a5, a6):
    n, din = input.shape
    dout = W.shape[1]

    support0 = _rowblock_matmul(input, W, block_rows=1000)

    h_A, A_bf16 = _rowblock_matmul_emit_bf16(A_nor, support0, block_rows=400)
    h_A2 = _rowblock_matmul(A_bf16, h_A.astype(jnp.bfloat16), block_rows=1000)
    h_A3 = _rowblock_matmul(A_bf16, h_A2.astype(jnp.bfloat16), block_rows=1000)
    h_s1 = _rowblock_matmul(P_sct1, support0, take_abs=True, block_rows=400)
    h_s2 = _rowblock_matmul(P_sct2, support0, take_abs=True, block_rows=400)
    h_s3 = _rowblock_matmul(P_sct3, support0, take_abs=True, block_rows=400)

    hs = (h_A, h_A2, h_A3, h_s1, h_s2, h_s3)
    hp, att = _epilogue(support0, hs, (a1, a2, a3, a4, a5, a6))
    return hp, att.reshape(n, 6, 1)
